# Initial kernel scaffold; baseline (speedup 1.0000x reference)
import jax
import jax.numpy as jnp
from jax.experimental import pallas as pl


def kernel(i_words, o_words, n_words, i_emb, o_emb):
    raise NotImplementedError("write your pallas kernel here")


# speedup vs baseline: 5.2574x; 5.2574x over previous
import functools

import jax
import jax.numpy as jnp
from jax import lax
from jax.experimental import pallas as pl
from jax.experimental.pallas import tpu as pltpu
from jax.experimental.pallas import tpu_sc as plsc

_NC = 2
_NS = 16
_LANES = 16
_SW = 32


def _transpose_sum(vecs, lane_iota):
    k = 1
    while len(vecs) > 1 or k <= _LANES // 2:
        mask = (lane_iota & k) != 0
        idx = lane_iota ^ k
        nxt = []
        for i in range(0, len(vecs), 2):
            a = vecs[i]
            b = vecs[i + 1] if i + 1 < len(vecs) else a
            first = jnp.where(mask, b, a)
            second = jnp.take_along_axis(jnp.where(mask, a, b), idx, axis=0)
            nxt.append(first + second)
        vecs = nxt
        k *= 2
    return vecs[0]


def _sc_scores(i_idx, on_idx_c, i_emb, o_emb, *, chunk):
    B = i_idx.shape[0]
    T = on_idx_c.shape[1]
    D = o_emb.shape[1]
    NW = _NC * _NS
    bpw = B // NW
    nch = bpw // chunk
    KD = D // _LANES
    C = chunk

    mesh = plsc.VectorSubcoreMesh(core_axis_name="c", subcore_axis_name="s")

    @functools.partial(
        pl.kernel,
        out_type=jax.ShapeDtypeStruct((NW * nch, C, _SW), jnp.float32),
        mesh=mesh,
        compiler_params=pltpu.CompilerParams(use_tc_tiling_on_sc=False),
        scratch_types=[
            pltpu.VMEM((C,), jnp.int32),
            pltpu.VMEM((T, C), jnp.int32),
            pltpu.VMEM((C, D), jnp.float32),
            pltpu.VMEM((T, C, D), jnp.float32),
            pltpu.VMEM((C, _SW), jnp.float32),
            pltpu.SemaphoreType.DMA,
        ],
    )
    def scores_kernel(i_idx_hbm, on_idx_hbm, i_emb_hbm, o_emb_hbm, out_hbm,
                      i_idx_v, on_idx_v, i_rows_v, on_rows_v, scores_v, sem):
        wid = lax.axis_index("s") * _NC + lax.axis_index("c")
        lane_iota = lax.iota(jnp.int32, _LANES)

        def chunk_body(ci, carry):
            base = wid * bpw + ci * C
            chunk_lin = wid * nch + ci
            pltpu.sync_copy(i_idx_hbm.at[pl.ds(base, C)], i_idx_v)
            pltpu.sync_copy(on_idx_hbm.at[chunk_lin], on_idx_v)
            cps = [pltpu.async_copy(i_emb_hbm.at[i_idx_v], i_rows_v, sem)]
            for t in range(T):
                cps.append(pltpu.async_copy(
                    o_emb_hbm.at[on_idx_v.at[t]], on_rows_v.at[t], sem))
            for cp in cps:
                cp.wait()

            def elem_body(e, inner):
                iv = [i_rows_v[e, pl.ds(kk * _LANES, _LANES)]
                      for kk in range(KD)]
                niv = [-v for v in iv]
                accs = []
                for t in range(T):
                    src = iv if t == 0 else niv
                    acc = src[0] * on_rows_v[t, e, pl.ds(0, _LANES)]
                    for kk in range(1, KD):
                        acc = acc + src[kk] * on_rows_v[t, e, pl.ds(kk * _LANES, _LANES)]
                    accs.append(acc)
                sA = _transpose_sum(accs[:_LANES], lane_iota)
                sB = _transpose_sum(accs[_LANES:], lane_iota)
                scores_v[e, pl.ds(0, _LANES)] = sA
                scores_v[e, pl.ds(_LANES, _LANES)] = sB
                return inner

            lax.fori_loop(0, C, elem_body, 0)
            pltpu.sync_copy(scores_v, out_hbm.at[chunk_lin])
            return carry

        lax.fori_loop(0, nch, chunk_body, 0)

    return scores_kernel(i_idx, on_idx_c, i_emb, o_emb)


def _make_loss_body(T):
    def _loss_body(s_ref, o_ref):
        x = s_ref[...]
        o_ref[...] = jnp.sum(jax.nn.log_sigmoid(x[:, :, :T]), keepdims=True)
    return _loss_body


def kernel(i_words, o_words, n_words, i_emb, o_emb):
    B, S = i_words.shape
    T = 1 + n_words.shape[1]
    NW = _NC * _NS
    C = 64
    bpw = B // NW
    nch = bpw // C
    i_idx = i_words.reshape(B)
    on_idx = jnp.concatenate([o_words, n_words], axis=1)
    on_idx_c = (on_idx.reshape(NW * nch, C, T)
                .transpose(0, 2, 1)
                .reshape(NW * nch, T, C))
    scores = _sc_scores(i_idx, on_idx_c, i_emb, o_emb, chunk=C)
    total = pl.pallas_call(
        _make_loss_body(T),
        out_shape=jax.ShapeDtypeStruct((1, 1, 1), jnp.float32),
    )(scores)
    return -total[0, 0, 0] / (B * S)
